# baseline (speedup 1.0000x reference)
import functools

import jax
import jax.numpy as jnp
from jax.experimental import pallas as pl
from jax.experimental.pallas import tpu as pltpu


def _fused_body(wr_ref, x_hbm, w1_hbm, w3_hbm, w2_hbm,
                logits_ref, out_hbm, acc_ref):
    B, L, D = x_hbm.shape
    E = wr_ref.shape[1]
    FFN = w1_hbm.shape[2]
    RLT = 512
    RNL = L // RLT
    LT = 512
    NL = L // LT
    K = 2

    def router_inner(x_ref):
        l = pl.program_id(0)
        part = jnp.sum(x_ref[...], axis=1)

        @pl.when(l == 0)
        def _init():
            acc_ref[...] = part

        @pl.when(l > 0)
        def _acc():
            acc_ref[...] += part

    pltpu.emit_pipeline(
        router_inner,
        grid=(RNL,),
        in_specs=[pl.BlockSpec((B, RLT, D), lambda l: (0, l, 0))],
    )(x_hbm)

    xm = acc_ref[...] * (1.0 / L)
    logits = jnp.dot(xm, wr_ref[...], preferred_element_type=jnp.float32)
    logits_ref[...] = logits
    p = jax.nn.softmax(logits, axis=-1)
    iota = jax.lax.broadcasted_iota(jnp.int32, p.shape, 1)
    i1 = jnp.argmax(p, axis=-1).astype(jnp.int32)
    m1 = jnp.max(p, axis=-1)
    p2 = jnp.where(iota == i1[:, None], -jnp.inf, p)
    i2 = jnp.argmax(p2, axis=-1).astype(jnp.int32)
    m2 = jnp.max(p2, axis=-1)

    def expert_of(b, k):
        first = jnp.where(b == 0, i1[0], i1[1])
        second = jnp.where(b == 0, i2[0], i2[1])
        return jnp.where(k == 0, first, second)

    def moe_inner(x_ref, w1_ref, w3_ref, w2_ref, out_ref):
        b = pl.program_id(0)
        k = pl.program_id(1)
        l = pl.program_id(2)
        x_t = x_ref[0]
        h1 = jnp.dot(x_t, w1_ref[0], preferred_element_type=jnp.float32)
        h3 = jnp.dot(x_t, w3_ref[0], preferred_element_type=jnp.float32)
        h = (h1 * jax.nn.sigmoid(h1)) * h3
        contrib = jnp.dot(h, w2_ref[0], preferred_element_type=jnp.float32)
        wgt1 = jnp.where(b == 0, m1[0], m1[1])
        wgt2 = jnp.where(b == 0, m2[0], m2[1])
        scale = jnp.where(k == 0, wgt1, wgt2)
        lslice = pl.ds(l * LT, LT)

        @pl.when(k == 0)
        def _first_expert():
            out_ref[0, lslice, :] = scale * contrib

        @pl.when(k == 1)
        def _second_expert():
            out_ref[0, lslice, :] += scale * contrib

    pltpu.emit_pipeline(
        moe_inner,
        grid=(B, K, NL),
        in_specs=[
            pl.BlockSpec((1, LT, D), lambda b, k, l: (b, l, 0)),
            pl.BlockSpec((1, D, FFN), lambda b, k, l: (expert_of(b, k), 0, 0)),
            pl.BlockSpec((1, D, FFN), lambda b, k, l: (expert_of(b, k), 0, 0)),
            pl.BlockSpec((1, FFN, D), lambda b, k, l: (expert_of(b, k), 0, 0)),
        ],
        out_specs=[pl.BlockSpec((1, L, D), lambda b, k, l: (b, 0, 0))],
    )(x_hbm, w1_hbm, w3_hbm, w2_hbm, out_hbm)


def kernel(hidden_states, Wr, w1, w2, w3):
    x = hidden_states
    B, L, D = x.shape
    E = Wr.shape[1]

    logits, out = pl.pallas_call(
        _fused_body,
        in_specs=[
            pl.BlockSpec(memory_space=pltpu.VMEM),
            pl.BlockSpec(memory_space=pltpu.HBM),
            pl.BlockSpec(memory_space=pltpu.HBM),
            pl.BlockSpec(memory_space=pltpu.HBM),
            pl.BlockSpec(memory_space=pltpu.HBM),
        ],
        out_specs=(
            pl.BlockSpec(memory_space=pltpu.VMEM),
            pl.BlockSpec(memory_space=pltpu.HBM),
        ),
        out_shape=(
            jax.ShapeDtypeStruct((B, E), jnp.float32),
            jax.ShapeDtypeStruct((B, L, D), jnp.float32),
        ),
        scratch_shapes=[pltpu.VMEM((B, D), jnp.float32)],
    )(Wr, x, w1, w3, w2)

    return (out, logits)


# speedup vs baseline: 1.0069x; 1.0069x over previous
import jax
import jax.numpy as jnp
from jax.experimental import pallas as pl
from jax.experimental.pallas import tpu as pltpu


def _router_body(x_ref, wr_ref, logits_ref, tkw_ref, tki_ref):
    x = x_ref[...]
    inv_l = 1.0 / x.shape[1]
    xm = jnp.sum(x, axis=1) * inv_l
    logits = jnp.dot(xm, wr_ref[...], preferred_element_type=jnp.float32)
    logits_ref[...] = logits
    p = jax.nn.softmax(logits, axis=-1)
    iota = jax.lax.broadcasted_iota(jnp.int32, p.shape, 1)
    i1 = jnp.argmax(p, axis=-1).astype(jnp.int32)
    m1 = jnp.max(p, axis=-1)
    p2 = jnp.where(iota == i1[:, None], -jnp.inf, p)
    i2 = jnp.argmax(p2, axis=-1).astype(jnp.int32)
    m2 = jnp.max(p2, axis=-1)
    tkw_ref[...] = jnp.concatenate([m1[:, None], m2[:, None]], axis=1)
    tki_ref[...] = jnp.concatenate([i1[:, None], i2[:, None]], axis=1)


def _moe_body(tki_ref, tkw_ref, x_ref, w1_ref, w3_ref, w2_ref, out_ref):
    b = pl.program_id(0)
    k = pl.program_id(1)
    l = pl.program_id(2)
    lt = x_ref.shape[1]
    x = x_ref[0]
    h1 = jnp.dot(x, w1_ref[0], preferred_element_type=jnp.float32)
    h3 = jnp.dot(x, w3_ref[0], preferred_element_type=jnp.float32)
    h = (h1 * jax.nn.sigmoid(h1)) * h3
    contrib = jnp.dot(h, w2_ref[0], preferred_element_type=jnp.float32)
    scale = tkw_ref[b, k]
    lslice = pl.ds(l * lt, lt)

    @pl.when(k == 0)
    def _first_expert():
        out_ref[0, lslice, :] = scale * contrib

    @pl.when(k == 1)
    def _second_expert():
        out_ref[0, lslice, :] += scale * contrib


def kernel(hidden_states, Wr, w1, w2, w3):
    x = hidden_states
    B, L, D = x.shape
    E = Wr.shape[1]
    FFN = w1.shape[2]
    K = 2
    LT = 512
    NL = L // LT

    logits, tkw, tki = pl.pallas_call(
        _router_body,
        out_shape=(
            jax.ShapeDtypeStruct((B, E), jnp.float32),
            jax.ShapeDtypeStruct((B, K), jnp.float32),
            jax.ShapeDtypeStruct((B, K), jnp.int32),
        ),
    )(x, Wr)

    grid_spec = pltpu.PrefetchScalarGridSpec(
        num_scalar_prefetch=2,
        grid=(B, K, NL),
        in_specs=[
            pl.BlockSpec((1, LT, D), lambda b, k, l, ti, tw: (b, l, 0)),
            pl.BlockSpec((1, D, FFN), lambda b, k, l, ti, tw: (ti[b, k], 0, 0)),
            pl.BlockSpec((1, D, FFN), lambda b, k, l, ti, tw: (ti[b, k], 0, 0)),
            pl.BlockSpec((1, FFN, D), lambda b, k, l, ti, tw: (ti[b, k], 0, 0)),
        ],
        out_specs=pl.BlockSpec((1, L, D), lambda b, k, l, ti, tw: (b, 0, 0)),
    )
    out = pl.pallas_call(
        _moe_body,
        grid_spec=grid_spec,
        out_shape=jax.ShapeDtypeStruct((B, L, D), jnp.float32),
    )(tki, tkw, x, w1, w3, w2)

    return (out, logits)
